# baseline (speedup 1.0000x reference)
import functools

import jax
import jax.numpy as jnp
from jax import lax
from jax.experimental import pallas as pl
from jax.experimental.pallas import tpu as pltpu
from jax.experimental.pallas import tpu_sc as plsc

_B, _T, _D = 32, 1024, 32
_K = 8192
_M = 256


def _argmin_body(z_ref, c_ref, code_ref, loss_ref):
    i = pl.program_id(0)
    steps_per_b = _T // _M
    z = z_ref[...]
    c = c_ref[...]
    s = lax.dot_general(z, c, (((1,), (1,)), ((), ())),
                        preferred_element_type=jnp.float32)
    ones = jnp.ones((1, _D), jnp.float32)
    c2 = lax.dot_general(ones, c * c, (((1,), (1,)), ((), ())),
                         preferred_element_type=jnp.float32,
                         precision=lax.Precision.HIGHEST)
    z2 = jnp.sum(z * z, axis=1, keepdims=True)
    dist = (z2 - 2.0 * s) + c2
    code_ref[...] = jnp.argmin(dist, axis=1).astype(jnp.int32)
    mins = jnp.min(dist, axis=1)
    step_sum = jnp.sum(mins)

    @pl.when(i == 0)
    def _():
        loss_ref[...] = jnp.zeros_like(loss_ref)

    b = i // steps_per_b
    col = lax.broadcasted_iota(jnp.int32, (1, _B), 1)
    loss_ref[...] = loss_ref[...] + jnp.where(col == b, step_sum, 0.0)

    @pl.when(i == (_B * _T) // _M - 1)
    def _():
        loss_ref[...] = loss_ref[...] * (1.0 / (_T * _D))


def _argmin_call(z_flat, codebook):
    grid = (_B * _T) // _M
    return pl.pallas_call(
        _argmin_body,
        grid=(grid,),
        in_specs=[
            pl.BlockSpec((_M, _D), lambda i: (i, 0)),
            pl.BlockSpec((_K, _D), lambda i: (0, 0)),
        ],
        out_specs=[
            pl.BlockSpec((_M,), lambda i: (i,)),
            pl.BlockSpec((1, _B), lambda i: (0, 0)),
        ],
        out_shape=[
            jax.ShapeDtypeStruct((_B * _T,), jnp.int32),
            jax.ShapeDtypeStruct((1, _B), jnp.float32),
        ],
    )(z_flat, codebook)


def _sc_gather(codebook, codes):
    info = plsc.get_sparse_core_info()
    nw = info.num_cores * info.num_subcores
    n = codes.shape[0]
    b_per_w = n // nw
    mesh = plsc.VectorSubcoreMesh(core_axis_name="c", subcore_axis_name="s")

    @functools.partial(
        pl.kernel,
        mesh=mesh,
        compiler_params=pltpu.CompilerParams(use_tc_tiling_on_sc=False),
        out_type=jax.ShapeDtypeStruct((n, _D), jnp.float32),
        scratch_types=[
            pltpu.VMEM((b_per_w,), jnp.int32),
            pltpu.VMEM((b_per_w, _D), jnp.float32),
            pltpu.SemaphoreType.DMA,
        ],
    )
    def k(table_hbm, idx_hbm, out_hbm, idx_v, rows_v, sem):
        wid = lax.axis_index("s") * info.num_cores + lax.axis_index("c")
        base = wid * b_per_w
        pltpu.sync_copy(idx_hbm.at[pl.ds(base, b_per_w)], idx_v)
        pltpu.async_copy(table_hbm.at[idx_v], rows_v, sem).wait()
        pltpu.sync_copy(rows_v, out_hbm.at[pl.ds(base, b_per_w)])

    return k(codebook, codes)


def kernel(z_e, codebook):
    z_flat = z_e.reshape(_B * _T, _D)
    code_flat, loss2d = _argmin_call(z_flat, codebook)
    z_q_flat = _sc_gather(codebook, code_flat)
    z_q = z_q_flat.reshape(_B, _T, _D)
    code = code_flat.reshape(_B, _T)
    loss = loss2d.reshape(_B)
    return z_q, code, loss, loss


# speedup vs baseline: 1.2079x; 1.2079x over previous
import functools

import jax
import jax.numpy as jnp
from jax import lax
from jax.experimental import pallas as pl
from jax.experimental.pallas import tpu as pltpu
from jax.experimental.pallas import tpu_sc as plsc

_B, _T, _D = 32, 1024, 32
_K = 8192
_M = 512


def _argmin_body(z_ref, c_ref, code_ref, loss_ref):
    i = pl.program_id(0)
    steps_per_b = _T // _M
    z = z_ref[...]
    c = c_ref[...]
    s = lax.dot_general(z, c, (((1,), (1,)), ((), ())),
                        preferred_element_type=jnp.float32)
    ones = jnp.ones((1, _D), jnp.float32)
    c2 = lax.dot_general(ones, c * c, (((1,), (1,)), ((), ())),
                         preferred_element_type=jnp.float32,
                         precision=lax.Precision.HIGHEST)
    z2 = jnp.sum(z * z, axis=1, keepdims=True)
    dist = (z2 - 2.0 * s) + c2
    code_ref[...] = jnp.argmin(dist, axis=1).astype(jnp.int32)
    mins = jnp.min(dist, axis=1)
    step_sum = jnp.sum(mins)

    @pl.when(i == 0)
    def _():
        loss_ref[...] = jnp.zeros_like(loss_ref)

    b = i // steps_per_b
    col = lax.broadcasted_iota(jnp.int32, (1, _B), 1)
    loss_ref[...] = loss_ref[...] + jnp.where(col == b, step_sum, 0.0)

    @pl.when(i == (_B * _T) // _M - 1)
    def _():
        loss_ref[...] = loss_ref[...] * (1.0 / (_T * _D))


def _argmin_call(z_flat, codebook):
    grid = (_B * _T) // _M
    return pl.pallas_call(
        _argmin_body,
        grid=(grid,),
        in_specs=[
            pl.BlockSpec((_M, _D), lambda i: (i, 0)),
            pl.BlockSpec((_K, _D), lambda i: (0, 0)),
        ],
        out_specs=[
            pl.BlockSpec((_M,), lambda i: (i,)),
            pl.BlockSpec((1, _B), lambda i: (0, 0)),
        ],
        out_shape=[
            jax.ShapeDtypeStruct((_B * _T,), jnp.int32),
            jax.ShapeDtypeStruct((1, _B), jnp.float32),
        ],
    )(z_flat, codebook)


def _sc_gather(codebook, codes):
    info = plsc.get_sparse_core_info()
    nw = info.num_cores * info.num_subcores
    n = codes.shape[0]
    b_per_w = n // nw
    mesh = plsc.VectorSubcoreMesh(core_axis_name="c", subcore_axis_name="s")

    @functools.partial(
        pl.kernel,
        mesh=mesh,
        compiler_params=pltpu.CompilerParams(use_tc_tiling_on_sc=False),
        out_type=jax.ShapeDtypeStruct((n, _D), jnp.float32),
        scratch_types=[
            pltpu.VMEM((b_per_w,), jnp.int32),
            pltpu.VMEM((b_per_w, _D), jnp.float32),
            pltpu.SemaphoreType.DMA,
        ],
    )
    def k(table_hbm, idx_hbm, out_hbm, idx_v, rows_v, sem):
        wid = lax.axis_index("s") * info.num_cores + lax.axis_index("c")
        base = wid * b_per_w
        pltpu.sync_copy(idx_hbm.at[pl.ds(base, b_per_w)], idx_v)
        pltpu.async_copy(table_hbm.at[idx_v], rows_v, sem).wait()
        pltpu.sync_copy(rows_v, out_hbm.at[pl.ds(base, b_per_w)])

    return k(codebook, codes)


def kernel(z_e, codebook):
    z_flat = z_e.reshape(_B * _T, _D)
    code_flat, loss2d = _argmin_call(z_flat, codebook)
    z_q_flat = _sc_gather(codebook, code_flat)
    z_q = z_q_flat.reshape(_B, _T, _D)
    code = code_flat.reshape(_B, _T)
    loss = loss2d.reshape(_B)
    return z_q, code, loss, loss


# speedup vs baseline: 1.2728x; 1.0538x over previous
import functools

import jax
import jax.numpy as jnp
from jax import lax
from jax.experimental import pallas as pl
from jax.experimental.pallas import tpu as pltpu
from jax.experimental.pallas import tpu_sc as plsc

_B, _T, _D = 32, 1024, 32
_K = 8192
_M = 1024


def _argmin_body(z_ref, c_ref, code_ref, loss_ref):
    i = pl.program_id(0)
    steps_per_b = _T // _M
    z = z_ref[...]
    c = c_ref[...]
    s = lax.dot_general(z, c, (((1,), (1,)), ((), ())),
                        preferred_element_type=jnp.float32)
    ones = jnp.ones((1, _D), jnp.float32)
    c2 = lax.dot_general(ones, c * c, (((1,), (1,)), ((), ())),
                         preferred_element_type=jnp.float32,
                         precision=lax.Precision.HIGHEST)
    z2 = jnp.sum(z * z, axis=1, keepdims=True)
    dist = (z2 - 2.0 * s) + c2
    code_ref[...] = jnp.argmin(dist, axis=1).astype(jnp.int32)
    mins = jnp.min(dist, axis=1)
    step_sum = jnp.sum(mins)

    @pl.when(i == 0)
    def _():
        loss_ref[...] = jnp.zeros_like(loss_ref)

    b = i // steps_per_b
    col = lax.broadcasted_iota(jnp.int32, (1, _B), 1)
    loss_ref[...] = loss_ref[...] + jnp.where(col == b, step_sum, 0.0)

    @pl.when(i == (_B * _T) // _M - 1)
    def _():
        loss_ref[...] = loss_ref[...] * (1.0 / (_T * _D))


def _argmin_call(z_flat, codebook):
    grid = (_B * _T) // _M
    return pl.pallas_call(
        _argmin_body,
        grid=(grid,),
        in_specs=[
            pl.BlockSpec((_M, _D), lambda i: (i, 0)),
            pl.BlockSpec((_K, _D), lambda i: (0, 0)),
        ],
        out_specs=[
            pl.BlockSpec((_M,), lambda i: (i,)),
            pl.BlockSpec((1, _B), lambda i: (0, 0)),
        ],
        out_shape=[
            jax.ShapeDtypeStruct((_B * _T,), jnp.int32),
            jax.ShapeDtypeStruct((1, _B), jnp.float32),
        ],
    )(z_flat, codebook)


def _sc_gather(codebook, codes):
    info = plsc.get_sparse_core_info()
    nw = info.num_cores * info.num_subcores
    n = codes.shape[0]
    b_per_w = n // nw
    mesh = plsc.VectorSubcoreMesh(core_axis_name="c", subcore_axis_name="s")

    @functools.partial(
        pl.kernel,
        mesh=mesh,
        compiler_params=pltpu.CompilerParams(use_tc_tiling_on_sc=False),
        out_type=jax.ShapeDtypeStruct((n, _D), jnp.float32),
        scratch_types=[
            pltpu.VMEM((b_per_w,), jnp.int32),
            pltpu.VMEM((b_per_w, _D), jnp.float32),
            pltpu.SemaphoreType.DMA,
        ],
    )
    def k(table_hbm, idx_hbm, out_hbm, idx_v, rows_v, sem):
        wid = lax.axis_index("s") * info.num_cores + lax.axis_index("c")
        base = wid * b_per_w
        pltpu.sync_copy(idx_hbm.at[pl.ds(base, b_per_w)], idx_v)
        pltpu.async_copy(table_hbm.at[idx_v], rows_v, sem).wait()
        pltpu.sync_copy(rows_v, out_hbm.at[pl.ds(base, b_per_w)])

    return k(codebook, codes)


def kernel(z_e, codebook):
    z_flat = z_e.reshape(_B * _T, _D)
    code_flat, loss2d = _argmin_call(z_flat, codebook)
    z_q_flat = _sc_gather(codebook, code_flat)
    z_q = z_q_flat.reshape(_B, _T, _D)
    code = code_flat.reshape(_B, _T)
    loss = loss2d.reshape(_B)
    return z_q, code, loss, loss
